# Initial kernel scaffold; baseline (speedup 1.0000x reference)
import jax
import jax.numpy as jnp
from jax.experimental import pallas as pl


def kernel(x, edge_index, batch_index, l0_w1, l0_b1, l0_w2, l0_b2, l1_w1, l1_b1, l1_w2, l1_b2, l2_w1, l2_b1, l2_w2, l2_b2, ro_w1, ro_b1, ro_w2, ro_b2):
    raise NotImplementedError("write your pallas kernel here")


# speedup vs baseline: 4.3223x; 4.3223x over previous
import functools

import jax
import jax.numpy as jnp
from jax import lax
from jax.experimental import pallas as pl
from jax.experimental.pallas import tpu as pltpu
from jax.experimental.pallas import tpu_sc as plsc

N = 10000
E = 320000
D = 128
H = 128
T = 10
G = 64

NC = 2
NS = 16
NW = NC * NS
EPW = E // NW
CH = 80
NCHUNK = EPW // CH
RPS = 624
TAIL = N - NS * RPS
ZR = 104
ZSLABS = RPS // ZR

_mesh = plsc.VectorSubcoreMesh(
    core_axis_name="c", subcore_axis_name="s", num_cores=NC, num_subcores=NS)


@functools.partial(
    pl.kernel,
    out_type=jax.ShapeDtypeStruct((NC, N, D), jnp.float32),
    mesh=_mesh,
    scratch_types=[
        pltpu.VMEM((CH,), jnp.int32),
        pltpu.VMEM((CH,), jnp.int32),
        pltpu.VMEM((CH, D), jnp.float32),
        pltpu.VMEM((ZR, D), jnp.float32),
        pltpu.VMEM_SHARED((N, D), jnp.float32),
        pltpu.SemaphoreType.DMA,
    ],
)
def _sc_agg(x_hbm, src_hbm, dst_hbm, out_hbm, sidx, didx, rows, zbuf, acc, sem):
    cid = lax.axis_index("c")
    sid = lax.axis_index("s")
    wid = sid * NC + cid

    zv = jnp.zeros((16,), jnp.float32)

    def _zrow(i, carry):
        zbuf[i // (D // 16), pl.ds((i % (D // 16)) * 16, 16)] = zv
        return carry

    lax.fori_loop(0, ZR * (D // 16), _zrow, 0)

    def _zslab(j, carry):
        pltpu.sync_copy(zbuf, acc.at[pl.ds(sid * RPS + j * ZR, ZR)])
        return carry

    lax.fori_loop(0, ZSLABS, _zslab, 0)

    @pl.when(sid == NS - 1)
    def _():
        pltpu.sync_copy(zbuf.at[pl.ds(0, TAIL)], acc.at[pl.ds(NS * RPS, TAIL)])

    plsc.subcore_barrier()

    ebase = wid * EPW

    def _edges(k, carry):
        off = pl.multiple_of(ebase + k * CH, 8)
        pltpu.sync_copy(src_hbm.at[pl.ds(off, CH)], sidx)
        pltpu.sync_copy(dst_hbm.at[pl.ds(off, CH)], didx)
        pltpu.async_copy(x_hbm.at[sidx], rows, sem).wait()
        pltpu.sync_copy(rows, acc.at[didx], add=True)
        return carry

    lax.fori_loop(0, NCHUNK, _edges, 0)
    plsc.subcore_barrier()

    pltpu.sync_copy(acc.at[pl.ds(sid * RPS, RPS)],
                    out_hbm.at[cid, pl.ds(sid * RPS, RPS)])

    @pl.when(sid == NS - 1)
    def _():
        pltpu.sync_copy(acc.at[pl.ds(NS * RPS, TAIL)],
                        out_hbm.at[cid, pl.ds(NS * RPS, TAIL)])


BN = 1000
_PREC = lax.Precision.HIGHEST


def _mlp_body(x_ref, p_ref, w1_ref, b1_ref, w2_ref, b2_ref, o_ref):
    h = x_ref[...] + p_ref[0] + p_ref[1]
    h = jnp.maximum(
        jnp.dot(h, w1_ref[...], precision=_PREC,
                preferred_element_type=jnp.float32) + b1_ref[...], 0.0)
    o_ref[...] = jnp.maximum(
        jnp.dot(h, w2_ref[...], precision=_PREC,
                preferred_element_type=jnp.float32) + b2_ref[...], 0.0)


_mlp = pl.pallas_call(
    _mlp_body,
    grid=(N // BN,),
    in_specs=[
        pl.BlockSpec((BN, D), lambda i: (i, 0)),
        pl.BlockSpec((NC, BN, D), lambda i: (0, i, 0)),
        pl.BlockSpec((D, H), lambda i: (0, 0)),
        pl.BlockSpec((1, H), lambda i: (0, 0)),
        pl.BlockSpec((H, H), lambda i: (0, 0)),
        pl.BlockSpec((1, H), lambda i: (0, 0)),
    ],
    out_specs=pl.BlockSpec((BN, H), lambda i: (i, 0)),
    out_shape=jax.ShapeDtypeStruct((N, H), jnp.float32),
)


def _final_body(x_ref, p_ref, w1_ref, b1_ref, w2_ref, b2_ref, bidx_ref,
                rw1_ref, rb1_ref, rw2_ref, rb2_ref, o_ref, acc_ref):
    i = pl.program_id(0)
    h = x_ref[...] + p_ref[0] + p_ref[1]
    h = jnp.maximum(
        jnp.dot(h, w1_ref[...], precision=_PREC,
                preferred_element_type=jnp.float32) + b1_ref[...], 0.0)
    h = jnp.maximum(
        jnp.dot(h, w2_ref[...], precision=_PREC,
                preferred_element_type=jnp.float32) + b2_ref[...], 0.0)
    b = bidx_ref[0, 0, :]
    onehot = (b[:, None] == lax.broadcasted_iota(jnp.int32, (1, G), 1)
              ).astype(jnp.float32)
    part = lax.dot_general(onehot, h, (((0,), (0,)), ((), ())),
                           precision=_PREC,
                           preferred_element_type=jnp.float32)

    @pl.when(i == 0)
    def _():
        acc_ref[...] = part

    @pl.when(i > 0)
    def _():
        acc_ref[...] += part

    @pl.when(i == N // BN - 1)
    def _():
        pooled = acc_ref[...]
        r = jnp.maximum(
            jnp.dot(pooled, rw1_ref[...], precision=_PREC,
                    preferred_element_type=jnp.float32) + rb1_ref[...], 0.0)
        o_ref[...] = jnp.dot(r, rw2_ref[...], precision=_PREC,
                             preferred_element_type=jnp.float32) + rb2_ref[...]


_final = pl.pallas_call(
    _final_body,
    grid=(N // BN,),
    in_specs=[
        pl.BlockSpec((BN, D), lambda i: (i, 0)),
        pl.BlockSpec((NC, BN, D), lambda i: (0, i, 0)),
        pl.BlockSpec((D, H), lambda i: (0, 0)),
        pl.BlockSpec((1, H), lambda i: (0, 0)),
        pl.BlockSpec((H, H), lambda i: (0, 0)),
        pl.BlockSpec((1, H), lambda i: (0, 0)),
        pl.BlockSpec((1, 1, BN), lambda i: (i, 0, 0)),
        pl.BlockSpec((H, H), lambda i: (0, 0)),
        pl.BlockSpec((1, H), lambda i: (0, 0)),
        pl.BlockSpec((H, T), lambda i: (0, 0)),
        pl.BlockSpec((1, T), lambda i: (0, 0)),
    ],
    out_specs=pl.BlockSpec((G, T), lambda i: (0, 0)),
    out_shape=jax.ShapeDtypeStruct((G, T), jnp.float32),
    scratch_shapes=[pltpu.VMEM((G, D), jnp.float32)],
)


def kernel(x, edge_index, batch_index,
           l0_w1, l0_b1, l0_w2, l0_b2,
           l1_w1, l1_b1, l1_w2, l1_b2,
           l2_w1, l2_b1, l2_w2, l2_b2,
           ro_w1, ro_b1, ro_w2, ro_b2):
    src = edge_index[0]
    dst = edge_index[1]
    bidx = batch_index.reshape(N // BN, 1, BN)

    p = _sc_agg(x, src, dst)
    h = _mlp(x, p, l0_w1, l0_b1.reshape(1, H), l0_w2, l0_b2.reshape(1, H))
    p = _sc_agg(h, src, dst)
    h = _mlp(h, p, l1_w1, l1_b1.reshape(1, H), l1_w2, l1_b2.reshape(1, H))
    p = _sc_agg(h, src, dst)
    out = _final(h, p, l2_w1, l2_b1.reshape(1, H), l2_w2, l2_b2.reshape(1, H),
                 bidx, ro_w1, ro_b1.reshape(1, H), ro_w2, ro_b2.reshape(1, T))
    return out


# speedup vs baseline: 11.1027x; 2.5687x over previous
import functools

import jax
import jax.numpy as jnp
from jax import lax
from jax.experimental import pallas as pl
from jax.experimental.pallas import tpu as pltpu
from jax.experimental.pallas import tpu_sc as plsc

N = 10000
E = 320000
D = 128
H = 128
T = 10
G = 64

NC = 2
NS = 16
NW = NC * NS
EPW = E // NW
CH = 40
NCHUNK = EPW // CH
NBUF = 5
RPS = 624
TAIL = N - NS * RPS
ZR = 8
ZSLABS = RPS // ZR

_mesh = plsc.VectorSubcoreMesh(
    core_axis_name="c", subcore_axis_name="s", num_cores=NC, num_subcores=NS)


@functools.partial(
    pl.kernel,
    out_type=jax.ShapeDtypeStruct((NC, N, D), jnp.float32),
    mesh=_mesh,
    scratch_types=[
        pltpu.VMEM((EPW,), jnp.int32),
        pltpu.VMEM((NBUF, CH), jnp.int32),
        pltpu.VMEM((NBUF, CH, D), jnp.float32),
        pltpu.VMEM((ZR, D), jnp.float32),
        pltpu.VMEM_SHARED((N, D), jnp.float32),
        pltpu.SemaphoreType.DMA,
        [pltpu.SemaphoreType.DMA] * NBUF,
        [pltpu.SemaphoreType.DMA] * NBUF,
    ],
)
def _sc_agg(x_hbm, src_hbm, dst_hbm, out_hbm, sidx, didx, rows, zbuf, acc,
            isem, gsems, dsems):
    cid = lax.axis_index("c")
    sid = lax.axis_index("s")
    wid = sid * NC + cid
    ebase = wid * EPW

    iload = pltpu.async_copy(
        src_hbm.at[pl.ds(pl.multiple_of(ebase, 8), EPW)], sidx, isem)

    zv = jnp.zeros((16,), jnp.float32)

    def _zrow(i, carry):
        zbuf[i // (D // 16), pl.ds((i % (D // 16)) * 16, 16)] = zv
        return carry

    lax.fori_loop(0, ZR * (D // 16), _zrow, 0)

    def _zslab(j, carry):
        pltpu.sync_copy(zbuf, acc.at[pl.ds(sid * RPS + j * ZR, ZR)])
        return carry

    lax.fori_loop(0, ZSLABS, _zslab, 0)

    @pl.when(sid == NS - 1)
    def _():
        pltpu.sync_copy(zbuf.at[pl.ds(0, TAIL)], acc.at[pl.ds(NS * RPS, TAIL)])

    iload.wait()
    plsc.subcore_barrier()

    def _dst_copy(k, b):
        off = pl.multiple_of(ebase + k * CH, 8)
        return pltpu.make_async_copy(dst_hbm.at[pl.ds(off, CH)], didx.at[b],
                                     dsems[b])

    def _dst_load(k, b):
        off = pl.multiple_of(ebase + k * CH, 8)
        pltpu.async_copy(dst_hbm.at[pl.ds(off, CH)], didx.at[b], dsems[b])

    def _sidx(k):
        return sidx.at[pl.ds(pl.multiple_of(k * CH, 8), CH)]

    for b in range(NBUF):
        _dst_load(b, b)
        pltpu.async_copy(x_hbm.at[_sidx(b)], rows.at[b], gsems[b])

    def _group(g, carry):
        for b in range(NBUF):
            k = g * NBUF + b
            pltpu.make_async_copy(x_hbm.at[_sidx(k)], rows.at[b],
                                  gsems[b]).wait()
            _dst_copy(k, b).wait()
            pltpu.sync_copy(rows.at[b], acc.at[didx.at[b]], add=True)

            @pl.when(k + NBUF < NCHUNK)
            def _():
                _dst_load(k + NBUF, b)
                pltpu.async_copy(x_hbm.at[_sidx(k + NBUF)], rows.at[b],
                                 gsems[b])
        return carry

    lax.fori_loop(0, NCHUNK // NBUF, _group, 0)
    plsc.subcore_barrier()

    pltpu.sync_copy(acc.at[pl.ds(sid * RPS, RPS)],
                    out_hbm.at[cid, pl.ds(sid * RPS, RPS)])

    @pl.when(sid == NS - 1)
    def _():
        pltpu.sync_copy(acc.at[pl.ds(NS * RPS, TAIL)],
                        out_hbm.at[cid, pl.ds(NS * RPS, TAIL)])


BN = 1000
_PREC = lax.Precision.HIGHEST


def _mlp_core(x_ref, p_ref, w1_ref, b1_ref, w2_ref, b2_ref):
    h = x_ref[...] + p_ref[0] + p_ref[1]
    h = jnp.maximum(
        jnp.dot(h, w1_ref[...], precision=_PREC,
                preferred_element_type=jnp.float32) + b1_ref[...], 0.0)
    return jnp.maximum(
        jnp.dot(h, w2_ref[...], precision=_PREC,
                preferred_element_type=jnp.float32) + b2_ref[...], 0.0)


def _mlp_body(x_ref, p_ref, w1_ref, b1_ref, w2_ref, b2_ref, o_ref):
    o_ref[...] = _mlp_core(x_ref, p_ref, w1_ref, b1_ref, w2_ref, b2_ref)


_mlp = pl.pallas_call(
    _mlp_body,
    grid=(N // BN,),
    in_specs=[
        pl.BlockSpec((BN, D), lambda i: (i, 0)),
        pl.BlockSpec((NC, BN, D), lambda i: (0, i, 0)),
        pl.BlockSpec((D, H), lambda i: (0, 0)),
        pl.BlockSpec((1, H), lambda i: (0, 0)),
        pl.BlockSpec((H, H), lambda i: (0, 0)),
        pl.BlockSpec((1, H), lambda i: (0, 0)),
    ],
    out_specs=pl.BlockSpec((BN, H), lambda i: (i, 0)),
    out_shape=jax.ShapeDtypeStruct((N, H), jnp.float32),
)


def _final_body(x_ref, p_ref, w1_ref, b1_ref, w2_ref, b2_ref, bidx_ref,
                rw1_ref, rb1_ref, rw2_ref, rb2_ref, o_ref, acc_ref):
    i = pl.program_id(0)
    h = _mlp_core(x_ref, p_ref, w1_ref, b1_ref, w2_ref, b2_ref)
    b = bidx_ref[0, 0, :]
    onehot = (b[:, None] == lax.broadcasted_iota(jnp.int32, (1, G), 1)
              ).astype(jnp.float32)
    part = lax.dot_general(onehot, h, (((0,), (0,)), ((), ())),
                           precision=_PREC,
                           preferred_element_type=jnp.float32)

    @pl.when(i == 0)
    def _():
        acc_ref[...] = part

    @pl.when(i > 0)
    def _():
        acc_ref[...] += part

    @pl.when(i == N // BN - 1)
    def _():
        pooled = acc_ref[...]
        r = jnp.maximum(
            jnp.dot(pooled, rw1_ref[...], precision=_PREC,
                    preferred_element_type=jnp.float32) + rb1_ref[...], 0.0)
        o_ref[...] = jnp.dot(r, rw2_ref[...], precision=_PREC,
                             preferred_element_type=jnp.float32) + rb2_ref[...]


_final = pl.pallas_call(
    _final_body,
    grid=(N // BN,),
    in_specs=[
        pl.BlockSpec((BN, D), lambda i: (i, 0)),
        pl.BlockSpec((NC, BN, D), lambda i: (0, i, 0)),
        pl.BlockSpec((D, H), lambda i: (0, 0)),
        pl.BlockSpec((1, H), lambda i: (0, 0)),
        pl.BlockSpec((H, H), lambda i: (0, 0)),
        pl.BlockSpec((1, H), lambda i: (0, 0)),
        pl.BlockSpec((1, 1, BN), lambda i: (i, 0, 0)),
        pl.BlockSpec((H, H), lambda i: (0, 0)),
        pl.BlockSpec((1, H), lambda i: (0, 0)),
        pl.BlockSpec((H, T), lambda i: (0, 0)),
        pl.BlockSpec((1, T), lambda i: (0, 0)),
    ],
    out_specs=pl.BlockSpec((G, T), lambda i: (0, 0)),
    out_shape=jax.ShapeDtypeStruct((G, T), jnp.float32),
    scratch_shapes=[pltpu.VMEM((G, D), jnp.float32)],
)


def kernel(x, edge_index, batch_index,
           l0_w1, l0_b1, l0_w2, l0_b2,
           l1_w1, l1_b1, l1_w2, l1_b2,
           l2_w1, l2_b1, l2_w2, l2_b2,
           ro_w1, ro_b1, ro_w2, ro_b2):
    src = edge_index[0]
    dst = edge_index[1]
    bidx = batch_index.reshape(N // BN, 1, BN)

    p = _sc_agg(x, src, dst)
    h = _mlp(x, p, l0_w1, l0_b1.reshape(1, H), l0_w2, l0_b2.reshape(1, H))
    p = _sc_agg(h, src, dst)
    h = _mlp(h, p, l1_w1, l1_b1.reshape(1, H), l1_w2, l1_b2.reshape(1, H))
    p = _sc_agg(h, src, dst)
    out = _final(h, p, l2_w1, l2_b1.reshape(1, H), l2_w2, l2_b2.reshape(1, H),
                 bidx, ro_w1, ro_b1.reshape(1, H), ro_w2, ro_b2.reshape(1, T))
    return out
